# Initial kernel scaffold; baseline (speedup 1.0000x reference)
import jax
import jax.numpy as jnp
from jax.experimental import pallas as pl


def kernel(x, edge_index, W1, b1, W2, b2):
    raise NotImplementedError("write your pallas kernel here")


# speedup vs baseline: 17.3378x; 17.3378x over previous
import functools

import jax
import jax.numpy as jnp
from jax import lax
from jax.experimental import pallas as pl
from jax.experimental.pallas import tpu as pltpu
from jax.experimental.pallas import tpu_sc as plsc

_NPAD = 10240
_CH = 128
_NW = 32
_LANES = 16
_ROWB = 1024
_SC_PARAMS = pltpu.CompilerParams(use_tc_tiling_on_sc=False)


def _fill(rows_v, ch, d, value):
    @pl.loop(0, ch)
    def _(r):
        @pl.loop(0, d // _LANES)
        def _(j):
            rows_v[r, pl.ds(j * _LANES, _LANES)] = jnp.full(
                (_LANES,), value, jnp.float32)


def _sc_degree(dst, npad, d):
    epad = dst.shape[0]
    cpw = epad // (_CH * _NW)
    mesh = plsc.VectorSubcoreMesh(core_axis_name="c", subcore_axis_name="s")

    @functools.partial(
        pl.kernel,
        out_type=jax.ShapeDtypeStruct((2, npad, d), jnp.float32),
        mesh=mesh,
        scratch_types=[
            pltpu.VMEM((1, _CH), jnp.int32),
            pltpu.VMEM((_CH, d), jnp.float32),
            pltpu.VMEM_SHARED((npad, d), jnp.float32),
        ],
        compiler_params=_SC_PARAMS,
    )
    def k(dst_hbm, out_hbm, dst_v, rows_v, acc_sh):
        c = lax.axis_index("c")
        s = lax.axis_index("s")
        w = c * 16 + s
        rpt = npad // 16

        _fill(rows_v, _CH, d, 0.0)

        @pl.loop(0, rpt // _CH)
        def _(t):
            pltpu.sync_copy(rows_v, acc_sh.at[pl.ds(s * rpt + t * _CH, _CH)])

        _fill(rows_v, _CH, d, 1.0)
        plsc.subcore_barrier()

        base = w * cpw * _CH

        @pl.loop(0, cpw)
        def _(t):
            pltpu.sync_copy(dst_hbm.at[pl.ds(base + t * _CH, _CH)],
                            dst_v.at[0])
            pltpu.sync_copy(rows_v, acc_sh.at[dst_v.at[0]], add=True)

        plsc.subcore_barrier()

        @pl.loop(0, rpt // _CH)
        def _(t):
            r0 = s * rpt + t * _CH
            pltpu.sync_copy(acc_sh.at[pl.ds(r0, _CH)],
                            out_hbm.at[c, pl.ds(r0, _CH)])

    return k(dst)


def _sc_edge_agg(table, src, dst):
    npad, d = table.shape
    epad = src.shape[0]
    cpw = epad // (_CH * _NW)
    mesh = plsc.VectorSubcoreMesh(core_axis_name="c", subcore_axis_name="s")

    @functools.partial(
        pl.kernel,
        out_type=jax.ShapeDtypeStruct((2, npad, d), jnp.float32),
        mesh=mesh,
        scratch_types=[
            pltpu.VMEM((1, _CH), jnp.int32),
            pltpu.VMEM((1, _CH), jnp.int32),
            pltpu.VMEM((_CH, d), jnp.float32),
            pltpu.VMEM_SHARED((npad, d), jnp.float32),
        ],
        compiler_params=_SC_PARAMS,
    )
    def k(table_hbm, src_hbm, dst_hbm, out_hbm, src_v, dst_v, rows_v, acc_sh):
        c = lax.axis_index("c")
        s = lax.axis_index("s")
        w = c * 16 + s
        rpt = npad // 16

        _fill(rows_v, _CH, d, 0.0)

        @pl.loop(0, rpt // _CH)
        def _(t):
            pltpu.sync_copy(rows_v, acc_sh.at[pl.ds(s * rpt + t * _CH, _CH)])

        plsc.subcore_barrier()

        base = w * cpw * _CH

        @pl.loop(0, cpw)
        def _(t):
            e0 = base + t * _CH
            pltpu.sync_copy(src_hbm.at[pl.ds(e0, _CH)], src_v.at[0])
            pltpu.sync_copy(dst_hbm.at[pl.ds(e0, _CH)], dst_v.at[0])
            pltpu.sync_copy(table_hbm.at[src_v.at[0]], rows_v)
            pltpu.sync_copy(rows_v, acc_sh.at[dst_v.at[0]], add=True)

        plsc.subcore_barrier()

        @pl.loop(0, rpt // _CH)
        def _(t):
            r0 = s * rpt + t * _CH
            pltpu.sync_copy(acc_sh.at[pl.ds(r0, _CH)],
                            out_hbm.at[c, pl.ds(r0, _CH)])

    return k(table, src, dst)


def _tc_matmul(x, w):
    n, kdim = x.shape
    m = w.shape[1]

    def body(x_ref, w_ref, o_ref):
        o_ref[...] = jnp.dot(x_ref[...], w_ref[...],
                             preferred_element_type=jnp.float32)

    return pl.pallas_call(
        body,
        grid=(n // _ROWB,),
        in_specs=[
            pl.BlockSpec((_ROWB, kdim), lambda i: (i, 0)),
            pl.BlockSpec((kdim, m), lambda i: (0, 0)),
        ],
        out_specs=pl.BlockSpec((_ROWB, m), lambda i: (i, 0)),
        out_shape=jax.ShapeDtypeStruct((n, m), jnp.float32),
    )(x, w)


def _tc_scale(deg_parts, h1):
    n, dh = h1.shape
    dd = deg_parts.shape[2]

    def body(dp_ref, h_ref, dinv_ref, hs_ref):
        deg = 1.0 + dp_ref[0, :, 0:1] + dp_ref[1, :, 0:1]
        dinv = lax.rsqrt(deg)
        dinv_ref[...] = dinv
        hs_ref[...] = h_ref[...] * dinv

    return pl.pallas_call(
        body,
        grid=(n // _ROWB,),
        in_specs=[
            pl.BlockSpec((2, _ROWB, dd), lambda i: (0, i, 0)),
            pl.BlockSpec((_ROWB, dh), lambda i: (i, 0)),
        ],
        out_specs=[
            pl.BlockSpec((_ROWB, 1), lambda i: (i, 0)),
            pl.BlockSpec((_ROWB, dh), lambda i: (i, 0)),
        ],
        out_shape=[
            jax.ShapeDtypeStruct((n, 1), jnp.float32),
            jax.ShapeDtypeStruct((n, dh), jnp.float32),
        ],
    )(deg_parts, h1)


def _tc_layer2(agg1, h1, dinv, b1, w2, n_real):
    n, dh = h1.shape
    dout = w2.shape[1]

    def body(ag_ref, h_ref, dv_ref, b_ref, w_ref, h2_ref, hs2_ref):
        i = pl.program_id(0)
        a = ag_ref[0] + ag_ref[1]
        dinv_b = dv_ref[...]
        pre = dinv_b * a + (dinv_b * dinv_b) * h_ref[...] + b_ref[...]
        out1 = jnp.maximum(pre, 0.0)
        row = lax.broadcasted_iota(jnp.int32, (_ROWB, 1), 0) + i * _ROWB
        out1 = jnp.where(row < n_real, out1, 0.0)
        h2 = jnp.dot(out1, w_ref[...], preferred_element_type=jnp.float32)
        h2_ref[...] = h2
        hs2_ref[...] = h2 * dinv_b

    return pl.pallas_call(
        body,
        grid=(n // _ROWB,),
        in_specs=[
            pl.BlockSpec((2, _ROWB, dh), lambda i: (0, i, 0)),
            pl.BlockSpec((_ROWB, dh), lambda i: (i, 0)),
            pl.BlockSpec((_ROWB, 1), lambda i: (i, 0)),
            pl.BlockSpec((1, dh), lambda i: (0, 0)),
            pl.BlockSpec((dh, dout), lambda i: (0, 0)),
        ],
        out_specs=[
            pl.BlockSpec((_ROWB, dout), lambda i: (i, 0)),
            pl.BlockSpec((_ROWB, dout), lambda i: (i, 0)),
        ],
        out_shape=[
            jax.ShapeDtypeStruct((n, dout), jnp.float32),
            jax.ShapeDtypeStruct((n, dout), jnp.float32),
        ],
    )(agg1, h1, dinv, b1, w2)


def _tc_final(agg2, h2, dinv, b2):
    n, dout = h2.shape

    def body(ag_ref, h_ref, dv_ref, b_ref, o_ref):
        a = ag_ref[0] + ag_ref[1]
        dinv_b = dv_ref[...]
        o = dinv_b * a + (dinv_b * dinv_b) * h_ref[...] + b_ref[...]
        m = jnp.max(o, axis=1, keepdims=True)
        e = jnp.exp(o - m)
        lse = jnp.log(jnp.sum(e, axis=1, keepdims=True)) + m
        o_ref[...] = o - lse

    return pl.pallas_call(
        body,
        grid=(n // _ROWB,),
        in_specs=[
            pl.BlockSpec((2, _ROWB, dout), lambda i: (0, i, 0)),
            pl.BlockSpec((_ROWB, dout), lambda i: (i, 0)),
            pl.BlockSpec((_ROWB, 1), lambda i: (i, 0)),
            pl.BlockSpec((1, dout), lambda i: (0, 0)),
        ],
        out_specs=pl.BlockSpec((_ROWB, dout), lambda i: (i, 0)),
        out_shape=jax.ShapeDtypeStruct((n, dout), jnp.float32),
    )(agg2, h2, dinv, b2)


def kernel(x, edge_index, W1, b1, W2, b2):
    n, d_in = x.shape
    e = edge_index.shape[1]

    cpw = -(-e // (_CH * _NW))
    epad = cpw * _CH * _NW
    src_p = jnp.concatenate(
        [edge_index[0], jnp.full((epad - e,), n, jnp.int32)])
    dst_p = jnp.concatenate(
        [edge_index[1], jnp.full((epad - e,), n, jnp.int32)])
    x_p = jnp.pad(x, ((0, _NPAD - n), (0, 0)))

    deg_parts = _sc_degree(dst_p, _NPAD, 16)
    h1 = _tc_matmul(x_p, W1)
    dinv, hs1 = _tc_scale(deg_parts, h1)
    agg1 = _sc_edge_agg(hs1, src_p, dst_p)
    h2, hs2 = _tc_layer2(agg1, h1, dinv,
                         b1.reshape(1, -1), W2, n)
    agg2 = _sc_edge_agg(hs2, src_p, dst_p)
    out = _tc_final(agg2, h2, dinv, b2.reshape(1, -1))
    return out[:n]


# speedup vs baseline: 23.5043x; 1.3557x over previous
import functools

import jax
import jax.numpy as jnp
from jax import lax
from jax.experimental import pallas as pl
from jax.experimental.pallas import tpu as pltpu
from jax.experimental.pallas import tpu_sc as plsc

_NPAD = 10240
_CH = 128
_NW = 32
_LANES = 16
_NB = 8
_GL = 4
_ROWB = 1024
_SC_PARAMS = pltpu.CompilerParams(use_tc_tiling_on_sc=False)


def _fill(buf, ch, d, value):
    @pl.loop(0, ch)
    def _(r):
        @pl.loop(0, d // _LANES)
        def _(j):
            buf[r, pl.ds(j * _LANES, _LANES)] = jnp.full(
                (_LANES,), value, jnp.float32)


def _sc_degree(dst2, npad, d):
    cpw = dst2.shape[0] // _NW
    win = 16
    mesh = plsc.VectorSubcoreMesh(core_axis_name="c", subcore_axis_name="s")

    @functools.partial(
        pl.kernel,
        out_type=jax.ShapeDtypeStruct((2, npad, d), jnp.float32),
        mesh=mesh,
        scratch_types=[
            pltpu.VMEM((cpw, _CH), jnp.int32),
            pltpu.VMEM((_CH, d), jnp.float32),
            pltpu.VMEM((_CH, d), jnp.float32),
            pltpu.VMEM_SHARED((npad, d), jnp.float32),
            pltpu.SemaphoreType.DMA,
            pltpu.SemaphoreType.DMA,
            pltpu.SemaphoreType.DMA,
            pltpu.SemaphoreType.DMA,
        ],
        compiler_params=_SC_PARAMS,
    )
    def k(dst_hbm, out_hbm, dst_v, zbuf, obuf, acc_sh, isem, zsem, ssem, osem):
        c = lax.axis_index("c")
        s = lax.axis_index("s")
        w = c * 16 + s
        rpt = npad // 16

        pltpu.async_copy(dst_hbm.at[pl.ds(w * cpw, cpw)], dst_v, isem)
        _fill(zbuf, _CH, d, 0.0)
        _fill(obuf, _CH, d, 1.0)
        for q in range(rpt // _CH):
            pltpu.async_copy(
                zbuf, acc_sh.at[pl.ds(s * rpt + q * _CH, _CH)], zsem)
        pltpu.make_async_copy(dst_hbm.at[pl.ds(w * cpw, cpw)], dst_v,
                              isem).wait()
        for q in range(rpt // _CH):
            pltpu.make_async_copy(
                zbuf, acc_sh.at[pl.ds(s * rpt + q * _CH, _CH)], zsem).wait()
        plsc.subcore_barrier()

        for t in range(win):
            pltpu.async_copy(obuf, acc_sh.at[dst_v.at[t]], ssem, add=True)

        @pl.loop(win, cpw)
        def _(t):
            pltpu.make_async_copy(obuf, acc_sh.at[dst_v.at[t]], ssem).wait()
            pltpu.async_copy(obuf, acc_sh.at[dst_v.at[t]], ssem, add=True)

        for t in range(win):
            pltpu.make_async_copy(obuf, acc_sh.at[dst_v.at[t]], ssem).wait()
        plsc.subcore_barrier()

        for q in range(rpt // _CH):
            r0 = s * rpt + q * _CH
            pltpu.async_copy(acc_sh.at[pl.ds(r0, _CH)],
                             out_hbm.at[c, pl.ds(r0, _CH)], osem)
        for q in range(rpt // _CH):
            r0 = s * rpt + q * _CH
            pltpu.make_async_copy(acc_sh.at[pl.ds(r0, _CH)],
                                  out_hbm.at[c, pl.ds(r0, _CH)], osem).wait()

    return k(dst2)


def _sc_edge_agg(table, src2, dst2):
    npad, d = table.shape
    cpw = src2.shape[0] // _NW
    assert cpw % _NB == 0 and cpw // _NB >= 2
    mesh = plsc.VectorSubcoreMesh(core_axis_name="c", subcore_axis_name="s")

    @functools.partial(
        pl.kernel,
        out_type=jax.ShapeDtypeStruct((2, npad, d), jnp.float32),
        mesh=mesh,
        scratch_types=(
            [pltpu.VMEM((cpw, _CH), jnp.int32)] * 2
            + [pltpu.VMEM((_CH, d), jnp.float32)] * _NB
            + [pltpu.VMEM_SHARED((npad, d), jnp.float32)]
            + [pltpu.SemaphoreType.DMA] * 4
            + [pltpu.SemaphoreType.DMA] * _NB
        ),
        compiler_params=_SC_PARAMS,
    )
    def k(table_hbm, src_hbm, dst_hbm, out_hbm, src_v, dst_v, *rest):
        rb = rest[:_NB]
        acc_sh = rest[_NB]
        isem, zsem, gsem, osem = rest[_NB + 1:_NB + 5]
        ssems = rest[_NB + 5:]
        c = lax.axis_index("c")
        s = lax.axis_index("s")
        w = c * 16 + s
        rpt = npad // 16

        pltpu.async_copy(src_hbm.at[pl.ds(w * cpw, cpw)], src_v, isem)
        pltpu.async_copy(dst_hbm.at[pl.ds(w * cpw, cpw)], dst_v, isem)
        _fill(rb[0], _CH, d, 0.0)
        for q in range(rpt // _CH):
            pltpu.async_copy(
                rb[0], acc_sh.at[pl.ds(s * rpt + q * _CH, _CH)], zsem)
        pltpu.make_async_copy(src_hbm.at[pl.ds(w * cpw, cpw)], src_v,
                              isem).wait()
        pltpu.make_async_copy(dst_hbm.at[pl.ds(w * cpw, cpw)], dst_v,
                              isem).wait()
        for q in range(rpt // _CH):
            pltpu.make_async_copy(
                rb[0], acc_sh.at[pl.ds(s * rpt + q * _CH, _CH)], zsem).wait()
        plsc.subcore_barrier()

        def issue_g(t, j):
            pltpu.async_copy(table_hbm.at[src_v.at[t]], rb[j], gsem)

        def wait_g(t, j):
            pltpu.make_async_copy(table_hbm.at[src_v.at[t]], rb[j],
                                  gsem).wait()

        def issue_s(t, j):
            pltpu.async_copy(rb[j], acc_sh.at[dst_v.at[t]], ssems[j],
                             add=True)

        def wait_s(t, j):
            pltpu.make_async_copy(rb[j], acc_sh.at[dst_v.at[t]],
                                  ssems[j]).wait()

        for j in range(_GL):
            issue_g(j, j)

        for j in range(_NB):
            wait_g(j, j)
            issue_s(j, j)
            u = j + _GL
            bu = u % _NB
            if u >= _NB:
                wait_s(u - _NB, bu)
            issue_g(u, bu)

        @pl.loop(1, cpw // _NB - 1)
        def _(t8):
            for j in range(_NB):
                t = t8 * _NB + j
                wait_g(t, j)
                issue_s(t, j)
                bu = (j + _GL) % _NB
                wait_s(t + _GL - _NB, bu)
                issue_g(t + _GL, bu)

        base = cpw - _NB
        for j in range(_NB):
            t = base + j
            wait_g(t, j)
            issue_s(t, j)
            u = t + _GL
            if u < cpw:
                bu = (j + _GL) % _NB
                wait_s(u - _NB, bu)
                issue_g(u, bu)

        for j in range(_NB):
            wait_s(base + j, j)
        plsc.subcore_barrier()

        for q in range(rpt // _CH):
            r0 = s * rpt + q * _CH
            pltpu.async_copy(acc_sh.at[pl.ds(r0, _CH)],
                             out_hbm.at[c, pl.ds(r0, _CH)], osem)
        for q in range(rpt // _CH):
            r0 = s * rpt + q * _CH
            pltpu.make_async_copy(acc_sh.at[pl.ds(r0, _CH)],
                                  out_hbm.at[c, pl.ds(r0, _CH)], osem).wait()

    return k(table, src2, dst2)


def _tc_matmul(x, w):
    n, kdim = x.shape
    m = w.shape[1]

    def body(x_ref, w_ref, o_ref):
        o_ref[...] = jnp.dot(x_ref[...], w_ref[...],
                             preferred_element_type=jnp.float32)

    return pl.pallas_call(
        body,
        grid=(n // _ROWB,),
        in_specs=[
            pl.BlockSpec((_ROWB, kdim), lambda i: (i, 0)),
            pl.BlockSpec((kdim, m), lambda i: (0, 0)),
        ],
        out_specs=pl.BlockSpec((_ROWB, m), lambda i: (i, 0)),
        out_shape=jax.ShapeDtypeStruct((n, m), jnp.float32),
    )(x, w)


def _tc_scale(deg_parts, h1):
    n, dh = h1.shape
    dd = deg_parts.shape[2]

    def body(dp_ref, h_ref, dinv_ref, hs_ref):
        deg = 1.0 + dp_ref[0, :, 0:1] + dp_ref[1, :, 0:1]
        dinv = lax.rsqrt(deg)
        dinv_ref[...] = dinv
        hs_ref[...] = h_ref[...] * dinv

    return pl.pallas_call(
        body,
        grid=(n // _ROWB,),
        in_specs=[
            pl.BlockSpec((2, _ROWB, dd), lambda i: (0, i, 0)),
            pl.BlockSpec((_ROWB, dh), lambda i: (i, 0)),
        ],
        out_specs=[
            pl.BlockSpec((_ROWB, 1), lambda i: (i, 0)),
            pl.BlockSpec((_ROWB, dh), lambda i: (i, 0)),
        ],
        out_shape=[
            jax.ShapeDtypeStruct((n, 1), jnp.float32),
            jax.ShapeDtypeStruct((n, dh), jnp.float32),
        ],
    )(deg_parts, h1)


def _tc_layer2(agg1, h1, dinv, b1, w2, n_real):
    n, dh = h1.shape
    dout = w2.shape[1]

    def body(ag_ref, h_ref, dv_ref, b_ref, w_ref, h2_ref, hs2_ref):
        i = pl.program_id(0)
        a = ag_ref[0] + ag_ref[1]
        dinv_b = dv_ref[...]
        pre = dinv_b * a + (dinv_b * dinv_b) * h_ref[...] + b_ref[...]
        out1 = jnp.maximum(pre, 0.0)
        row = lax.broadcasted_iota(jnp.int32, (_ROWB, 1), 0) + i * _ROWB
        out1 = jnp.where(row < n_real, out1, 0.0)
        h2 = jnp.dot(out1, w_ref[...], preferred_element_type=jnp.float32)
        h2_ref[...] = h2
        hs2_ref[...] = h2 * dinv_b

    return pl.pallas_call(
        body,
        grid=(n // _ROWB,),
        in_specs=[
            pl.BlockSpec((2, _ROWB, dh), lambda i: (0, i, 0)),
            pl.BlockSpec((_ROWB, dh), lambda i: (i, 0)),
            pl.BlockSpec((_ROWB, 1), lambda i: (i, 0)),
            pl.BlockSpec((1, dh), lambda i: (0, 0)),
            pl.BlockSpec((dh, dout), lambda i: (0, 0)),
        ],
        out_specs=[
            pl.BlockSpec((_ROWB, dout), lambda i: (i, 0)),
            pl.BlockSpec((_ROWB, dout), lambda i: (i, 0)),
        ],
        out_shape=[
            jax.ShapeDtypeStruct((n, dout), jnp.float32),
            jax.ShapeDtypeStruct((n, dout), jnp.float32),
        ],
    )(agg1, h1, dinv, b1, w2)


def _tc_final(agg2, h2, dinv, b2):
    n, dout = h2.shape

    def body(ag_ref, h_ref, dv_ref, b_ref, o_ref):
        a = ag_ref[0] + ag_ref[1]
        dinv_b = dv_ref[...]
        o = dinv_b * a + (dinv_b * dinv_b) * h_ref[...] + b_ref[...]
        m = jnp.max(o, axis=1, keepdims=True)
        e = jnp.exp(o - m)
        lse = jnp.log(jnp.sum(e, axis=1, keepdims=True)) + m
        o_ref[...] = o - lse

    return pl.pallas_call(
        body,
        grid=(n // _ROWB,),
        in_specs=[
            pl.BlockSpec((2, _ROWB, dout), lambda i: (0, i, 0)),
            pl.BlockSpec((_ROWB, dout), lambda i: (i, 0)),
            pl.BlockSpec((_ROWB, 1), lambda i: (i, 0)),
            pl.BlockSpec((1, dout), lambda i: (0, 0)),
        ],
        out_specs=pl.BlockSpec((_ROWB, dout), lambda i: (i, 0)),
        out_shape=jax.ShapeDtypeStruct((n, dout), jnp.float32),
    )(agg2, h2, dinv, b2)


def kernel(x, edge_index, W1, b1, W2, b2):
    n, d_in = x.shape
    e = edge_index.shape[1]

    cpw = -(-e // (_CH * _NW))
    cpw = -(-cpw // _NB) * _NB
    epad = cpw * _CH * _NW
    src_p = jnp.concatenate(
        [edge_index[0], jnp.full((epad - e,), n, jnp.int32)]).reshape(-1, _CH)
    dst_p = jnp.concatenate(
        [edge_index[1], jnp.full((epad - e,), n, jnp.int32)]).reshape(-1, _CH)
    x_p = jnp.pad(x, ((0, _NPAD - n), (0, 0)))

    deg_parts = _sc_degree(dst_p, _NPAD, 16)
    h1 = _tc_matmul(x_p, W1)
    dinv, hs1 = _tc_scale(deg_parts, h1)
    agg1 = _sc_edge_agg(hs1, src_p, dst_p)
    h2, hs2 = _tc_layer2(agg1, h1, dinv,
                         b1.reshape(1, -1), W2, n)
    agg2 = _sc_edge_agg(hs2, src_p, dst_p)
    out = _tc_final(agg2, h2, dinv, b2.reshape(1, -1))
    return out[:n]
